# baseline (speedup 1.0000x reference)
import functools
import numpy as np
import jax
import jax.numpy as jnp
from jax import lax
from jax.experimental import pallas as pl
from jax.experimental.pallas import tpu as pltpu

_B = 128
_N = 32768
_L = 128
_NR = _N // _L
_CH = 32
_STEPS = _NR // _CH

_KE0 = (0xBDFB82F1, 0x07B3B635)
_KE1 = (0x8C1266AC, 0x45A3D6BE)


def _np_rotl(x, r):
    return ((x << np.uint32(r)) | (x >> np.uint32(32 - r))).astype(np.uint32)


def _np_threefry_bits(k, lo):
    ks0, ks1 = np.uint32(k[0]), np.uint32(k[1])
    ks2 = np.uint32(ks0 ^ ks1 ^ np.uint32(0x1BD11BDA))
    rots = [13, 15, 26, 6, 17, 29, 16, 24]
    x0 = np.full_like(lo, ks0)
    x1 = (lo + ks1).astype(np.uint32)

    def four(x0, x1, rs):
        for r in rs:
            x0 = (x0 + x1).astype(np.uint32)
            x1 = _np_rotl(x1, r) ^ x0
        return x0, x1

    x0, x1 = four(x0, x1, rots[:4])
    x0 = (x0 + ks1).astype(np.uint32); x1 = (x1 + ks2 + np.uint32(1)).astype(np.uint32)
    x0, x1 = four(x0, x1, rots[4:])
    x0 = (x0 + ks2).astype(np.uint32); x1 = (x1 + ks0 + np.uint32(2)).astype(np.uint32)
    x0, x1 = four(x0, x1, rots[:4])
    x0 = (x0 + ks0).astype(np.uint32); x1 = (x1 + ks1 + np.uint32(3)).astype(np.uint32)
    x0, x1 = four(x0, x1, rots[4:])
    x0 = (x0 + ks1).astype(np.uint32); x1 = (x1 + ks2 + np.uint32(4)).astype(np.uint32)
    x0, x1 = four(x0, x1, rots[:4])
    x0 = (x0 + ks2).astype(np.uint32); x1 = (x1 + ks0 + np.uint32(5)).astype(np.uint32)
    return x0 ^ x1


@functools.lru_cache(maxsize=1)
def _gumbel_tables():
    n = _B * _N
    cnt = np.arange(n, dtype=np.uint32)
    tiny = np.float32(np.finfo(np.float32).tiny)

    def gum(kd):
        bits = _np_threefry_bits(kd, cnt)
        fl = ((bits >> np.uint32(9)) | np.uint32(0x3F800000)).view(np.float32)
        u = np.maximum(tiny, fl - np.float32(1.0))
        g = -np.log(-np.log(u))
        return g.reshape(_B, _NR, _L)

    return np.concatenate([gum(_KE0), gum(_KE1)], axis=2)


def _block_argmax(v, nmat):
    m = jnp.max(v, axis=1, keepdims=True)
    big = jnp.int32(np.iinfo(np.int32).max)
    idx = jnp.min(jnp.where(v == m, nmat, big), axis=1, keepdims=True)
    return m, idx


def _body(x_ref, g_ref, w_ref, b_ref, out_ref,
          v0_s, i0_s, v1_s, i1_s, v2_s, i2_s):
    step = pl.program_id(0)
    w0 = w_ref[0, 0]
    w1 = w_ref[0, 1]
    w2 = w_ref[0, 2]
    bias = b_ref[0, 0]

    r = lax.broadcasted_iota(jnp.int32, (3 * _L, _L), 0)
    c = lax.broadcasted_iota(jnp.int32, (3 * _L, _L), 1)
    rm = r % 3
    wsel = jnp.where(rm == 0, w0, jnp.where(rm == 1, w1, w2))
    S = jnp.where(r // 3 == c, wsel, jnp.float32(0.0))

    x2 = x_ref[...].reshape(_B * _CH, 3 * _L)
    lin = jnp.dot(x2, S, preferred_element_type=jnp.float32) + bias

    rr = lax.broadcasted_iota(jnp.int32, (_B * _CH, _L), 0)
    jj = lax.broadcasted_iota(jnp.int32, (_B * _CH, _L), 1)
    ic = rr % _CH
    n = (step * _CH + ic) * _L + jj

    gc = g_ref[...]
    g0 = gc[:, :, :_L].reshape(_B * _CH, _L)
    g1 = gc[:, :, _L:].reshape(_B * _CH, _L)

    flat = (_B, _CH * _L)
    nmat = n.reshape(flat)
    m0, x0i = _block_argmax(lin.reshape(flat), nmat)
    m1, x1i = _block_argmax((lin + g0).reshape(flat), nmat)
    m2, x2i = _block_argmax((lin + g1).reshape(flat), nmat)

    @pl.when(step == 0)
    def _init():
        v0_s[...], i0_s[...] = m0, x0i
        v1_s[...], i1_s[...] = m1, x1i
        v2_s[...], i2_s[...] = m2, x2i

    @pl.when(step != 0)
    def _merge():
        for m, idx, v_s, i_s in ((m0, x0i, v0_s, i0_s),
                                 (m1, x1i, v1_s, i1_s),
                                 (m2, x2i, v2_s, i2_s)):
            old_v = v_s[...]
            take = m > old_v
            v_s[...] = jnp.where(take, m, old_v)
            i_s[...] = jnp.where(take, idx, i_s[...])

    @pl.when(step == _STEPS - 1)
    def _emit():
        best = i0_s[...]
        c0 = i1_s[...]
        c1 = i2_s[...]
        out_ref[...] = jnp.where(c0 == best, c0, c1)


def kernel(X, W, b):
    Xr = X.reshape(_B, _NR, 3 * _L)
    b2 = b.reshape(1, 1)
    gt = _gumbel_tables()
    out = pl.pallas_call(
        _body,
        grid=(_STEPS,),
        in_specs=[
            pl.BlockSpec((_B, _CH, 3 * _L), lambda s: (0, s, 0)),
            pl.BlockSpec((_B, _CH, 2 * _L), lambda s: (0, s, 0)),
            pl.BlockSpec((1, 3), lambda s: (0, 0)),
            pl.BlockSpec((1, 1), lambda s: (0, 0)),
        ],
        out_specs=pl.BlockSpec((_B, 1), lambda s: (0, 0)),
        out_shape=jax.ShapeDtypeStruct((_B, 1), jnp.int32),
        scratch_shapes=[
            pltpu.VMEM((_B, 1), jnp.float32), pltpu.VMEM((_B, 1), jnp.int32),
            pltpu.VMEM((_B, 1), jnp.float32), pltpu.VMEM((_B, 1), jnp.int32),
            pltpu.VMEM((_B, 1), jnp.float32), pltpu.VMEM((_B, 1), jnp.int32),
        ],
        compiler_params=pltpu.CompilerParams(
            dimension_semantics=("arbitrary",),
        ),
    )(Xr, jnp.asarray(gt), W, b2)
    return out.reshape(_B)
